# baseline (speedup 1.0000x reference)
import functools

import jax
import jax.numpy as jnp
from jax import lax
from jax.experimental import pallas as pl
from jax.experimental.pallas import tpu as pltpu
from jax.experimental.pallas import tpu_sc as plsc

NUM_CORES = 2
NUM_SUBCORES = 16
NUM_WORKERS = NUM_CORES * NUM_SUBCORES
CHUNK = 128
GROUP = 4
UNROLL = 8
PAD = 129


@jax.jit
def _sc_embedding_gather(idx2d, table):
    rows, chunk = idx2d.shape
    vocab, dim = table.shape
    hist = rows // 32
    rows_per_w = rows // NUM_WORKERS
    n_groups = rows_per_w // GROUP
    assert n_groups % 2 == 0 and n_groups * GROUP == rows_per_w
    mesh = plsc.VectorSubcoreMesh(core_axis_name="c", subcore_axis_name="s")

    @functools.partial(
        pl.kernel,
        mesh=mesh,
        out_type=jax.ShapeDtypeStruct((hist, 4, 32, 8, CHUNK), jnp.float32),
        compiler_params=pltpu.CompilerParams(
            use_tc_tiling_on_sc=False, needs_layout_passes=False),
        scratch_types=[
            pltpu.VMEM((rows_per_w, chunk), jnp.int32),
            pltpu.VMEM((GROUP * CHUNK, dim), jnp.float32),
            pltpu.VMEM((GROUP * CHUNK, dim), jnp.float32),
            pltpu.VMEM((GROUP * 32, PAD), jnp.float32),
            pltpu.VMEM((GROUP * 32, PAD), jnp.float32),
            pltpu.VMEM((8, CHUNK), jnp.float32),
            pltpu.SemaphoreType.DMA,
            pltpu.SemaphoreType.DMA,
            pltpu.SemaphoreType.DMA,
            pltpu.SemaphoreType.DMA,
        ],
    )
    def k(idx_hbm, table_hbm, out_hbm, idx_v, rows0, rows1, t0, t1, dummy_v,
          gsem0, gsem1, ssem0, ssem1):
        wid = lax.axis_index("s") * NUM_CORES + lax.axis_index("c")
        u0 = wid * rows_per_w
        pltpu.sync_copy(idx_hbm.at[pl.ds(u0, rows_per_w)], idx_v)

        iota16 = lax.iota(jnp.int32, 16)
        row_lo_j = tuple(iota16 + (j * 32) for j in range(GROUP))
        row_hi_j = tuple(iota16 + (j * 32 + 16) for j in range(GROUP))
        kvec = tuple(iota16 * 0 + kk for kk in range(UNROLL))

        def fire(g, buf, sem):
            for j in range(GROUP):
                pltpu.async_copy(
                    table_hbm.at[idx_v.at[g * GROUP + j]],
                    buf.at[pl.ds(j * CHUNK, CHUNK)],
                    sem,
                )

        def drain(buf, sem):
            for j in range(GROUP):
                pltpu.make_async_copy(
                    table_hbm.at[pl.ds(0, CHUNK)],
                    buf.at[pl.ds(j * CHUNK, CHUNK)],
                    sem,
                ).wait()

        def transpose(rows, t):
            for j in range(GROUP):

                def tok_body(dci, carry):
                    dc0 = dci * UNROLL
                    b_v = jnp.full((16,), dc0, jnp.int32)
                    for kk in range(UNROLL):
                        row = j * CHUNK + dc0 + kk
                        col_v = b_v + kvec[kk]
                        x0 = rows[row, pl.ds(0, 16)]
                        x1 = rows[row, pl.ds(16, 16)]
                        plsc.store_scatter(t, [row_lo_j[j], col_v], x0)
                        plsc.store_scatter(t, [row_hi_j[j], col_v], x1)
                    return carry

                lax.fori_loop(0, CHUNK // UNROLL, tok_body, 0)

        def fire_stores(g, t, sem):
            for j in range(GROUP):
                u = u0 + g * GROUP + j
                h = lax.bitwise_or(
                    lax.shift_left(lax.shift_right_logical(u, 8), 3),
                    lax.bitwise_and(u, 7))
                tc = lax.bitwise_and(lax.shift_right_logical(u, 3), 31)
                for tr in range(4):
                    pltpu.async_copy(
                        t.at[pl.ds(j * 32 + tr * 8, 8), pl.ds(0, CHUNK)],
                        out_hbm.at[h, tr, tc],
                        sem,
                    )

        def drain_stores(t, sem):
            del t
            for _ in range(GROUP):
                for tr in range(4):
                    pltpu.make_async_copy(
                        dummy_v,
                        out_hbm.at[0, tr, 0],
                        sem,
                    ).wait()

        fire(0, rows0, gsem0)

        def body(t2, carry):
            g = 2 * t2
            fire(g + 1, rows1, gsem1)
            drain(rows0, gsem0)

            @pl.when(t2 > 0)
            def _():
                drain_stores(t0, ssem0)

            transpose(rows0, t0)
            fire_stores(g, t0, ssem0)

            @pl.when(t2 < n_groups // 2 - 1)
            def _():
                fire(g + 2, rows0, gsem0)

            drain(rows1, gsem1)

            @pl.when(t2 > 0)
            def _():
                drain_stores(t1, ssem1)

            transpose(rows1, t1)
            fire_stores(g + 1, t1, ssem1)
            return carry

        lax.fori_loop(0, n_groups // 2, body, 0)
        drain_stores(t0, ssem0)
        drain_stores(t1, ssem1)

    return k(idx2d, table)


def kernel(words_as_ids, embedding_weight):
    batch, hist = words_as_ids.shape
    dim = embedding_weight.shape[1]
    idx2d = (words_as_ids.reshape(batch // CHUNK, CHUNK, hist // 8, 8)
             .transpose(2, 0, 3, 1)
             .reshape(-1, CHUNK))
    out5d = _sc_embedding_gather(idx2d, embedding_weight)
    return out5d.transpose(2, 4, 0, 1, 3).reshape(batch, hist, dim)
